# baseline (speedup 1.0000x reference)
import functools
import math

import jax
import jax.numpy as jnp
from jax import lax
from jax.experimental import pallas as pl
from jax.experimental.pallas import tpu as pltpu
from jax.experimental.pallas import tpu_sc as plsc

_E = 160000
_N = 10000
_D = 16
_H = 256
_SEG = 10000

_NC = 2
_NS = 16
_NW = _NC * _NS
_BPW = _E // _NW
_GCH = 125
_NCH = _BPW // _GCH
_RPS = _SEG // _NS

@functools.cache
def _sc_mesh():
    return plsc.VectorSubcoreMesh(core_axis_name="c", subcore_axis_name="s")


def _gather_body(nodes_hbm, src_hbm, out_hbm, idx_v, rows_v, sem):
    wid = lax.axis_index("s") * _NC + lax.axis_index("c")
    pltpu.sync_copy(src_hbm.at[wid], idx_v)

    def fire(g, c):
        pltpu.async_copy(nodes_hbm.at[idx_v.at[g]], rows_v.at[g], sem)
        return c

    lax.fori_loop(0, _NCH, fire, 0)

    def drain(g, c):
        pltpu.make_async_copy(nodes_hbm.at[idx_v.at[g]], rows_v.at[g], sem).wait()
        return c

    lax.fori_loop(0, _NCH, drain, 0)
    pltpu.sync_copy(rows_v, out_hbm.at[wid])


@jax.jit
def _sc_gather(node_features, src_perm):
    src = src_perm.reshape(_NW, _NCH, _GCH).astype(jnp.int32)
    out = pl.kernel(
        _gather_body,
        out_type=jax.ShapeDtypeStruct((_NW, _NCH, _GCH, _D), jnp.float32),
        mesh=_sc_mesh(),
        scratch_types=[
            pltpu.VMEM((_NCH, _GCH), jnp.int32),
            pltpu.VMEM((_NCH, _GCH, _D), jnp.float32),
            pltpu.SemaphoreType.DMA,
        ],
        compiler_params=pltpu.CompilerParams(use_tc_tiling_on_sc=False),
    )(node_features, src)
    return out.reshape(_E // 8, 128)


_BE = 3200
_BR = _BE // 8


def _tc_body(es_ref, x1p_ref, attrp_ref, w1_ref, w2_ref, r_ref, s_ref, efp_ref):
    h = lax.dot_general(es_ref[...], w1_ref[...], (((0,), (0,)), ((), ())),
                        preferred_element_type=jnp.float32)
    h = jnp.maximum(h, 0.0)
    w = jnp.dot(h, w2_ref[...], preferred_element_type=jnp.float32)
    x1p = x1p_ref[...]
    pieces = []
    for s in range(8):
        xs = x1p[:, s * _D:(s + 1) * _D]
        x1e = jnp.dot(xs, r_ref[...], preferred_element_type=jnp.float32)
        t = x1e * w[s * _BR:(s + 1) * _BR, :]
        pieces.append(jnp.dot(t, s_ref[...], preferred_element_type=jnp.float32))
    efp_ref[...] = jnp.concatenate(pieces, axis=1) * attrp_ref[...]


@jax.jit
def _tc_fused(edge_scalars, x1p, edge_attr, W1, W2):
    j = jnp.arange(_H)
    r_mat = (j[None, :] // _D == jnp.arange(_D)[:, None]).astype(jnp.float32)
    s_mat = (j[:, None] % _D == jnp.arange(_D)[None, :]).astype(jnp.float32)
    es_t = edge_scalars.T
    W1 = W1 * (1.0 / math.sqrt(3.0))
    attrp = jnp.repeat(_pack_perm(edge_attr.reshape(-1)) * (1.0 / 256.0),
                       _D).reshape(_E // 8, 128)
    grid = _E // _BE
    return pl.pallas_call(
        _tc_body,
        grid=(grid,),
        in_specs=[
            pl.BlockSpec((3, _BE), lambda i: (0, i)),
            pl.BlockSpec((_BR, 128), lambda i: (i, 0)),
            pl.BlockSpec((_BR, 128), lambda i: (i, 0)),
            pl.BlockSpec((3, _H), lambda i: (0, 0)),
            pl.BlockSpec((_H, _H), lambda i: (0, 0)),
            pl.BlockSpec((_D, _H), lambda i: (0, 0)),
            pl.BlockSpec((_H, _D), lambda i: (0, 0)),
        ],
        out_specs=pl.BlockSpec((_BR, 128), lambda i: (i, 0)),
        out_shape=jax.ShapeDtypeStruct((_E // 8, 128), jnp.float32),
        compiler_params=pltpu.CompilerParams(
            dimension_semantics=("arbitrary",)),
    )(es_t, x1p, attrp, W1, W2, r_mat, s_mat)


def _pack_perm(idx):
    return idx.reshape(_E // _BE, 8, _BR).transpose(0, 2, 1).reshape(-1)


def _scatter_body(ef_hbm, dst_hbm, zero_hbm, out_hbm, idx_v, rows_v, acc_sh, sem):
    cid = lax.axis_index("c")
    sid = lax.axis_index("s")
    wid = sid * _NC + cid
    pltpu.sync_copy(zero_hbm.at[pl.ds(sid * _RPS, _RPS)],
                    acc_sh.at[pl.ds(sid * _RPS, _RPS)])
    plsc.subcore_barrier()
    pltpu.sync_copy(ef_hbm.at[wid], rows_v)
    pltpu.sync_copy(dst_hbm.at[wid], idx_v)

    def fire(g, c):
        pltpu.async_copy(rows_v.at[g], acc_sh.at[idx_v.at[g]], sem, add=True)
        return c

    lax.fori_loop(0, _NCH, fire, 0)

    def drain(g, c):
        pltpu.make_async_copy(rows_v.at[g], acc_sh.at[idx_v.at[g]], sem).wait()
        return c

    lax.fori_loop(0, _NCH, drain, 0)
    plsc.subcore_barrier()
    pltpu.sync_copy(acc_sh.at[pl.ds(sid * _RPS, _RPS)],
                    out_hbm.at[cid, pl.ds(sid * _RPS, _RPS)])


@jax.jit
def _sc_scatter(efp, dst_perm):
    dst = dst_perm.reshape(_NW, _NCH, _GCH).astype(jnp.int32)
    efr = efp.reshape(_NW, _NCH, _GCH, _D)
    zero = jnp.zeros((_SEG, _D), jnp.float32)
    partials = pl.kernel(
        _scatter_body,
        out_type=jax.ShapeDtypeStruct((_NC, _SEG, _D), jnp.float32),
        mesh=_sc_mesh(),
        scratch_types=[
            pltpu.VMEM((_NCH, _GCH), jnp.int32),
            pltpu.VMEM((_NCH, _GCH, _D), jnp.float32),
            pltpu.VMEM_SHARED((_SEG, _D), jnp.float32),
            pltpu.SemaphoreType.DMA,
        ],
        compiler_params=pltpu.CompilerParams(use_tc_tiling_on_sc=False),
    )(efr, dst, zero)
    return partials[0] + partials[1]


def kernel(node_features, edge_src, edge_dst, edge_attr, edge_scalars, W1, W2):
    x1p = _sc_gather(node_features, _pack_perm(edge_src))
    efp = _tc_fused(edge_scalars, x1p, edge_attr, W1, W2)
    return _sc_scatter(efp, _pack_perm(edge_dst))


# speedup vs baseline: 1.0396x; 1.0396x over previous
import functools
import math

import jax
import jax.numpy as jnp
from jax import lax
from jax.experimental import pallas as pl
from jax.experimental.pallas import tpu as pltpu
from jax.experimental.pallas import tpu_sc as plsc

_E = 160000
_N = 10000
_D = 16
_H = 256
_SEG = 10000

_NC = 2
_NS = 16
_NW = _NC * _NS
_BPW = _E // _NW
_GCH = 125
_NCH = _BPW // _GCH
_RPS = _SEG // _NS

@functools.cache
def _sc_mesh():
    return plsc.VectorSubcoreMesh(core_axis_name="c", subcore_axis_name="s")


def _gather_body(nodes_hbm, src_hbm, out_hbm, idx_v, rows_v, sem):
    wid = lax.axis_index("s") * _NC + lax.axis_index("c")
    pltpu.sync_copy(src_hbm.at[wid], idx_v)

    def fire(g, c):
        pltpu.async_copy(nodes_hbm.at[idx_v.at[g]], rows_v.at[g], sem)
        return c

    lax.fori_loop(0, _NCH, fire, 0)

    def drain(g, c):
        pltpu.make_async_copy(nodes_hbm.at[idx_v.at[g]], rows_v.at[g], sem).wait()
        return c

    lax.fori_loop(0, _NCH, drain, 0)
    pltpu.sync_copy(rows_v, out_hbm.at[wid])


@jax.jit
def _sc_gather(node_features, src_perm):
    src = src_perm.reshape(_NW, _NCH, _GCH).astype(jnp.int32)
    out = pl.kernel(
        _gather_body,
        out_type=jax.ShapeDtypeStruct((_NW, _NCH, _GCH, _D), jnp.float32),
        mesh=_sc_mesh(),
        scratch_types=[
            pltpu.VMEM((_NCH, _GCH), jnp.int32),
            pltpu.VMEM((_NCH, _GCH, _D), jnp.float32),
            pltpu.SemaphoreType.DMA,
        ],
        compiler_params=pltpu.CompilerParams(use_tc_tiling_on_sc=False),
    )(node_features, src)
    return out.reshape(_E // 8, 128)


_BE = 3200
_BR = _BE // 8


def _tc_body(es_ref, x1p_ref, attr_ref, w1_ref, w2_ref, r_ref, s_ref, efp_ref):
    h = lax.dot_general(es_ref[...], w1_ref[...], (((0,), (0,)), ((), ())),
                        preferred_element_type=jnp.float32)
    h = jnp.maximum(h, 0.0)
    w = jnp.dot(h, w2_ref[...], preferred_element_type=jnp.float32)
    scale_row = jnp.full((1, _D), 1.0 / 256.0, jnp.float32)
    attr_bc = lax.dot_general(attr_ref[0], scale_row, (((0,), (0,)), ((), ())),
                              preferred_element_type=jnp.float32)
    x1p = x1p_ref[...]
    pieces = []
    for s in range(8):
        xs = x1p[:, s * _D:(s + 1) * _D]
        x1e = jnp.dot(xs, r_ref[...], preferred_element_type=jnp.float32)
        t = x1e * w[s * _BR:(s + 1) * _BR, :]
        ef_s = jnp.dot(t, s_ref[...], preferred_element_type=jnp.float32)
        pieces.append(ef_s * attr_bc[s * _BR:(s + 1) * _BR, :])
    efp_ref[...] = jnp.concatenate(pieces, axis=1)


@jax.jit
def _tc_fused(edge_scalars, x1p, edge_attr, W1, W2):
    j = jnp.arange(_H)
    r_mat = (j[None, :] // _D == jnp.arange(_D)[:, None]).astype(jnp.float32)
    s_mat = (j[:, None] % _D == jnp.arange(_D)[None, :]).astype(jnp.float32)
    es_t = edge_scalars.T
    W1 = W1 * (1.0 / math.sqrt(3.0))
    attr_rows = edge_attr.reshape(_E // _BE, 1, _BE)
    grid = _E // _BE
    return pl.pallas_call(
        _tc_body,
        grid=(grid,),
        in_specs=[
            pl.BlockSpec((3, _BE), lambda i: (0, i)),
            pl.BlockSpec((_BR, 128), lambda i: (i, 0)),
            pl.BlockSpec((1, 1, _BE), lambda i: (i, 0, 0)),
            pl.BlockSpec((3, _H), lambda i: (0, 0)),
            pl.BlockSpec((_H, _H), lambda i: (0, 0)),
            pl.BlockSpec((_D, _H), lambda i: (0, 0)),
            pl.BlockSpec((_H, _D), lambda i: (0, 0)),
        ],
        out_specs=pl.BlockSpec((_BR, 128), lambda i: (i, 0)),
        out_shape=jax.ShapeDtypeStruct((_E // 8, 128), jnp.float32),
        compiler_params=pltpu.CompilerParams(
            dimension_semantics=("arbitrary",)),
    )(es_t, x1p, attr_rows, W1, W2, r_mat, s_mat)


def _pack_perm(idx):
    return idx.reshape(_E // _BE, 8, _BR).transpose(0, 2, 1).reshape(-1)


def _scatter_body(ef_hbm, dst_hbm, zero_hbm, out_hbm, idx_v, rows_v, acc_sh, sem):
    cid = lax.axis_index("c")
    sid = lax.axis_index("s")
    wid = sid * _NC + cid
    pltpu.sync_copy(zero_hbm.at[pl.ds(sid * _RPS, _RPS)],
                    acc_sh.at[pl.ds(sid * _RPS, _RPS)])
    plsc.subcore_barrier()
    pltpu.sync_copy(ef_hbm.at[wid], rows_v)
    pltpu.sync_copy(dst_hbm.at[wid], idx_v)

    def fire(g, c):
        pltpu.async_copy(rows_v.at[g], acc_sh.at[idx_v.at[g]], sem, add=True)
        return c

    lax.fori_loop(0, _NCH, fire, 0)

    def drain(g, c):
        pltpu.make_async_copy(rows_v.at[g], acc_sh.at[idx_v.at[g]], sem).wait()
        return c

    lax.fori_loop(0, _NCH, drain, 0)
    plsc.subcore_barrier()
    pltpu.sync_copy(acc_sh.at[pl.ds(sid * _RPS, _RPS)],
                    out_hbm.at[cid, pl.ds(sid * _RPS, _RPS)])


@jax.jit
def _sc_scatter(efp, dst_perm):
    dst = dst_perm.reshape(_NW, _NCH, _GCH).astype(jnp.int32)
    efr = efp.reshape(_NW, _NCH, _GCH, _D)
    zero = jnp.zeros((_SEG, _D), jnp.float32)
    partials = pl.kernel(
        _scatter_body,
        out_type=jax.ShapeDtypeStruct((_NC, _SEG, _D), jnp.float32),
        mesh=_sc_mesh(),
        scratch_types=[
            pltpu.VMEM((_NCH, _GCH), jnp.int32),
            pltpu.VMEM((_NCH, _GCH, _D), jnp.float32),
            pltpu.VMEM_SHARED((_SEG, _D), jnp.float32),
            pltpu.SemaphoreType.DMA,
        ],
        compiler_params=pltpu.CompilerParams(use_tc_tiling_on_sc=False),
    )(efr, dst, zero)
    return partials[0] + partials[1]


def kernel(node_features, edge_src, edge_dst, edge_attr, edge_scalars, W1, W2):
    x1p = _sc_gather(node_features, _pack_perm(edge_src))
    efp = _tc_fused(edge_scalars, x1p, edge_attr, W1, W2)
    return _sc_scatter(efp, _pack_perm(edge_dst))


# speedup vs baseline: 1.1044x; 1.0623x over previous
import functools
import math

import jax
import jax.numpy as jnp
from jax import lax
from jax.experimental import pallas as pl
from jax.experimental.pallas import tpu as pltpu
from jax.experimental.pallas import tpu_sc as plsc

_E = 160000
_N = 10000
_D = 16
_H = 256
_SEG = 10000

_NC = 2
_NS = 16
_NW = _NC * _NS
_BPW = _E // _NW
_GCH = 125
_NCH = _BPW // _GCH
_RPS = _SEG // _NS

@functools.cache
def _sc_mesh():
    return plsc.VectorSubcoreMesh(core_axis_name="c", subcore_axis_name="s")


def _gather_body(nodes_hbm, src_hbm, out_hbm, idx_v, rows_v, sem):
    wid = lax.axis_index("s") * _NC + lax.axis_index("c")
    pltpu.sync_copy(src_hbm.at[wid], idx_v)

    def fire(g, c):
        pltpu.async_copy(nodes_hbm.at[idx_v.at[g]], rows_v.at[g], sem)
        return c

    lax.fori_loop(0, _NCH, fire, 0)

    def drain(g, c):
        pltpu.make_async_copy(nodes_hbm.at[idx_v.at[g]], rows_v.at[g], sem).wait()
        return c

    lax.fori_loop(0, _NCH, drain, 0)
    pltpu.sync_copy(rows_v, out_hbm.at[wid])


@jax.jit
def _sc_gather(node_features, src_perm):
    src = src_perm.reshape(_NW, _NCH, _GCH).astype(jnp.int32)
    out = pl.kernel(
        _gather_body,
        out_type=jax.ShapeDtypeStruct((_NW, _NCH, _GCH, _D), jnp.float32),
        mesh=_sc_mesh(),
        scratch_types=[
            pltpu.VMEM((_NCH, _GCH), jnp.int32),
            pltpu.VMEM((_NCH, _GCH, _D), jnp.float32),
            pltpu.SemaphoreType.DMA,
        ],
        compiler_params=pltpu.CompilerParams(use_tc_tiling_on_sc=False),
    )(node_features, src)
    return out.reshape(_E // 8, 128)


_BE = 6400
_BR = _BE // 8


def _tc_body(es_ref, x1p_ref, attr_ref, w1_ref, w2_ref, r_ref, s_ref, efp_ref):
    h = lax.dot_general(es_ref[...], w1_ref[...], (((0,), (0,)), ((), ())),
                        preferred_element_type=jnp.float32)
    h = jnp.maximum(h, 0.0)
    w = jnp.dot(h, w2_ref[...], preferred_element_type=jnp.float32)
    scale_row = jnp.full((1, _D), 1.0 / 256.0, jnp.float32)
    attr_bc = lax.dot_general(attr_ref[0], scale_row, (((0,), (0,)), ((), ())),
                              preferred_element_type=jnp.float32)
    x1p = x1p_ref[...]
    pieces = []
    for s in range(8):
        xs = x1p[:, s * _D:(s + 1) * _D]
        x1e = jnp.dot(xs, r_ref[...], preferred_element_type=jnp.float32)
        t = x1e * w[s * _BR:(s + 1) * _BR, :]
        ef_s = jnp.dot(t, s_ref[...], preferred_element_type=jnp.float32)
        pieces.append(ef_s * attr_bc[s * _BR:(s + 1) * _BR, :])
    efp_ref[...] = jnp.concatenate(pieces, axis=1)


@jax.jit
def _tc_fused(edge_scalars, x1p, edge_attr, W1, W2):
    j = jnp.arange(_H)
    r_mat = (j[None, :] // _D == jnp.arange(_D)[:, None]).astype(jnp.float32)
    s_mat = (j[:, None] % _D == jnp.arange(_D)[None, :]).astype(jnp.float32)
    es_t = edge_scalars.T
    W1 = W1 * (1.0 / math.sqrt(3.0))
    attr_rows = edge_attr.reshape(_E // _BE, 1, _BE)
    grid = _E // _BE
    return pl.pallas_call(
        _tc_body,
        grid=(grid,),
        in_specs=[
            pl.BlockSpec((3, _BE), lambda i: (0, i)),
            pl.BlockSpec((_BR, 128), lambda i: (i, 0)),
            pl.BlockSpec((1, 1, _BE), lambda i: (i, 0, 0)),
            pl.BlockSpec((3, _H), lambda i: (0, 0)),
            pl.BlockSpec((_H, _H), lambda i: (0, 0)),
            pl.BlockSpec((_D, _H), lambda i: (0, 0)),
            pl.BlockSpec((_H, _D), lambda i: (0, 0)),
        ],
        out_specs=pl.BlockSpec((_BR, 128), lambda i: (i, 0)),
        out_shape=jax.ShapeDtypeStruct((_E // 8, 128), jnp.float32),
        compiler_params=pltpu.CompilerParams(
            dimension_semantics=("arbitrary",)),
    )(es_t, x1p, attr_rows, W1, W2, r_mat, s_mat)


def _pack_perm(idx):
    return idx.reshape(_E // _BE, 8, _BR).transpose(0, 2, 1).reshape(-1)


def _scatter_body(ef_hbm, dst_hbm, zero_hbm, out_hbm, idx_v, rows_v, acc_sh, sem):
    cid = lax.axis_index("c")
    sid = lax.axis_index("s")
    wid = sid * _NC + cid
    pltpu.sync_copy(zero_hbm.at[pl.ds(sid * _RPS, _RPS)],
                    acc_sh.at[pl.ds(sid * _RPS, _RPS)])
    plsc.subcore_barrier()
    pltpu.sync_copy(ef_hbm.at[wid], rows_v)
    pltpu.sync_copy(dst_hbm.at[wid], idx_v)

    def fire(g, c):
        pltpu.async_copy(rows_v.at[g], acc_sh.at[idx_v.at[g]], sem, add=True)
        return c

    lax.fori_loop(0, _NCH, fire, 0)

    def drain(g, c):
        pltpu.make_async_copy(rows_v.at[g], acc_sh.at[idx_v.at[g]], sem).wait()
        return c

    lax.fori_loop(0, _NCH, drain, 0)
    plsc.subcore_barrier()
    pltpu.sync_copy(acc_sh.at[pl.ds(sid * _RPS, _RPS)],
                    out_hbm.at[cid, pl.ds(sid * _RPS, _RPS)])


@jax.jit
def _sc_scatter(efp, dst_perm):
    dst = dst_perm.reshape(_NW, _NCH, _GCH).astype(jnp.int32)
    efr = efp.reshape(_NW, _NCH, _GCH, _D)
    zero = jnp.zeros((_SEG, _D), jnp.float32)
    partials = pl.kernel(
        _scatter_body,
        out_type=jax.ShapeDtypeStruct((_NC, _SEG, _D), jnp.float32),
        mesh=_sc_mesh(),
        scratch_types=[
            pltpu.VMEM((_NCH, _GCH), jnp.int32),
            pltpu.VMEM((_NCH, _GCH, _D), jnp.float32),
            pltpu.VMEM_SHARED((_SEG, _D), jnp.float32),
            pltpu.SemaphoreType.DMA,
        ],
        compiler_params=pltpu.CompilerParams(use_tc_tiling_on_sc=False),
    )(efr, dst, zero)
    return partials[0] + partials[1]


def kernel(node_features, edge_src, edge_dst, edge_attr, edge_scalars, W1, W2):
    x1p = _sc_gather(node_features, _pack_perm(edge_src))
    efp = _tc_fused(edge_scalars, x1p, edge_attr, W1, W2)
    return _sc_scatter(efp, _pack_perm(edge_dst))


# speedup vs baseline: 1.1203x; 1.0144x over previous
import functools
import math

import jax
import jax.numpy as jnp
from jax import lax
from jax.experimental import pallas as pl
from jax.experimental.pallas import tpu as pltpu
from jax.experimental.pallas import tpu_sc as plsc

_E = 160000
_N = 10000
_D = 16
_H = 256
_SEG = 10000

_NC = 2
_NS = 16
_NW = _NC * _NS
_BPW = _E // _NW
_GCH = 125
_NCH = _BPW // _GCH
_RPS = _SEG // _NS

@functools.cache
def _sc_mesh():
    return plsc.VectorSubcoreMesh(core_axis_name="c", subcore_axis_name="s")


def _gather_body(nodes_hbm, src_hbm, out_hbm, idx_v, rows_v, sem):
    nch = src_hbm.shape[1]
    wid = lax.axis_index("s") * _NC + lax.axis_index("c")
    pltpu.sync_copy(src_hbm.at[wid], idx_v)

    def fire(g, c):
        pltpu.async_copy(nodes_hbm.at[idx_v.at[g]], rows_v.at[g], sem)
        return c

    lax.fori_loop(0, nch, fire, 0)

    def drain(g, c):
        pltpu.make_async_copy(nodes_hbm.at[idx_v.at[g]], rows_v.at[g], sem).wait()
        return c

    lax.fori_loop(0, nch, drain, 0)
    pltpu.sync_copy(rows_v, out_hbm.at[wid])


@jax.jit
def _sc_gather(node_features, src_perm):
    e_loc = src_perm.shape[0]
    nch = e_loc // _NW // _GCH
    src = src_perm.reshape(_NW, nch, _GCH).astype(jnp.int32)
    out = pl.kernel(
        _gather_body,
        out_type=jax.ShapeDtypeStruct((_NW, nch, _GCH, _D), jnp.float32),
        mesh=_sc_mesh(),
        scratch_types=[
            pltpu.VMEM((nch, _GCH), jnp.int32),
            pltpu.VMEM((nch, _GCH, _D), jnp.float32),
            pltpu.SemaphoreType.DMA,
        ],
        compiler_params=pltpu.CompilerParams(use_tc_tiling_on_sc=False),
    )(node_features, src)
    return out.reshape(e_loc // 8, 128)


_BE = 16000
_BR = _BE // 8


def _tc_body(es_ref, x1p_ref, attr_ref, w1_ref, w2_ref, r_ref, s_ref, efp_ref):
    h = lax.dot_general(es_ref[...], w1_ref[...], (((0,), (0,)), ((), ())),
                        preferred_element_type=jnp.float32)
    h = jnp.maximum(h, 0.0)
    w = jnp.dot(h, w2_ref[...], preferred_element_type=jnp.float32)
    scale_row = jnp.full((1, _D), 1.0 / 256.0, jnp.float32)
    attr_bc = lax.dot_general(attr_ref[0], scale_row, (((0,), (0,)), ((), ())),
                              preferred_element_type=jnp.float32)
    x1p = x1p_ref[...]
    pieces = []
    for s in range(8):
        xs = x1p[:, s * _D:(s + 1) * _D]
        x1e = jnp.dot(xs, r_ref[...], preferred_element_type=jnp.float32)
        t = x1e * w[s * _BR:(s + 1) * _BR, :]
        ef_s = jnp.dot(t, s_ref[...], preferred_element_type=jnp.float32)
        pieces.append(ef_s * attr_bc[s * _BR:(s + 1) * _BR, :])
    efp_ref[...] = jnp.concatenate(pieces, axis=1)


@jax.jit
def _tc_fused(edge_scalars, x1p, edge_attr, W1, W2):
    j = jnp.arange(_H)
    r_mat = (j[None, :] // _D == jnp.arange(_D)[:, None]).astype(jnp.float32)
    s_mat = (j[:, None] % _D == jnp.arange(_D)[None, :]).astype(jnp.float32)
    e_loc = x1p.shape[0] * 8
    es_t = edge_scalars.T
    W1 = W1 * (1.0 / math.sqrt(3.0))
    attr_rows = edge_attr.reshape(e_loc // _BE, 1, _BE)
    grid = e_loc // _BE
    return pl.pallas_call(
        _tc_body,
        grid=(grid,),
        in_specs=[
            pl.BlockSpec((3, _BE), lambda i: (0, i)),
            pl.BlockSpec((_BR, 128), lambda i: (i, 0)),
            pl.BlockSpec((1, 1, _BE), lambda i: (i, 0, 0)),
            pl.BlockSpec((3, _H), lambda i: (0, 0)),
            pl.BlockSpec((_H, _H), lambda i: (0, 0)),
            pl.BlockSpec((_D, _H), lambda i: (0, 0)),
            pl.BlockSpec((_H, _D), lambda i: (0, 0)),
        ],
        out_specs=pl.BlockSpec((_BR, 128), lambda i: (i, 0)),
        out_shape=jax.ShapeDtypeStruct((e_loc // 8, 128), jnp.float32),
        compiler_params=pltpu.CompilerParams(
            dimension_semantics=("arbitrary",)),
    )(es_t, x1p, attr_rows, W1, W2, r_mat, s_mat)


def _pack_perm(idx):
    return idx.reshape(idx.shape[0] // _BE, 8, _BR).transpose(0, 2, 1).reshape(-1)


def _scatter_body(ef_hbm, dst_hbm, zero_hbm, out_hbm, idx_v, rows_v, acc_sh, sem):
    nch = dst_hbm.shape[1]
    cid = lax.axis_index("c")
    sid = lax.axis_index("s")
    wid = sid * _NC + cid
    pltpu.sync_copy(zero_hbm.at[pl.ds(sid * _RPS, _RPS)],
                    acc_sh.at[pl.ds(sid * _RPS, _RPS)])
    plsc.subcore_barrier()
    pltpu.sync_copy(ef_hbm.at[wid], rows_v)
    pltpu.sync_copy(dst_hbm.at[wid], idx_v)

    def fire(g, c):
        pltpu.async_copy(rows_v.at[g], acc_sh.at[idx_v.at[g]], sem, add=True)
        return c

    lax.fori_loop(0, nch, fire, 0)

    def drain(g, c):
        pltpu.make_async_copy(rows_v.at[g], acc_sh.at[idx_v.at[g]], sem).wait()
        return c

    lax.fori_loop(0, nch, drain, 0)
    plsc.subcore_barrier()
    pltpu.sync_copy(acc_sh.at[pl.ds(sid * _RPS, _RPS)],
                    out_hbm.at[cid, pl.ds(sid * _RPS, _RPS)])


@jax.jit
def _sc_scatter(efp, dst_perm):
    e_loc = dst_perm.shape[0]
    nch = e_loc // _NW // _GCH
    dst = dst_perm.reshape(_NW, nch, _GCH).astype(jnp.int32)
    efr = efp.reshape(_NW, nch, _GCH, _D)
    zero = jnp.zeros((_SEG, _D), jnp.float32)
    partials = pl.kernel(
        _scatter_body,
        out_type=jax.ShapeDtypeStruct((_NC, _SEG, _D), jnp.float32),
        mesh=_sc_mesh(),
        scratch_types=[
            pltpu.VMEM((nch, _GCH), jnp.int32),
            pltpu.VMEM((nch, _GCH, _D), jnp.float32),
            pltpu.VMEM_SHARED((_SEG, _D), jnp.float32),
            pltpu.SemaphoreType.DMA,
        ],
        compiler_params=pltpu.CompilerParams(use_tc_tiling_on_sc=False),
    )(efr, dst, zero)
    return partials[0] + partials[1]


def kernel(node_features, edge_src, edge_dst, edge_attr, edge_scalars, W1, W2):
    half = _E // 2
    out = None
    for lo in (0, half):
        sl = slice(lo, lo + half)
        x1p = _sc_gather(node_features, _pack_perm(edge_src[sl]))
        efp = _tc_fused(edge_scalars[sl], x1p, edge_attr[sl], W1, W2)
        part = _sc_scatter(efp, _pack_perm(edge_dst[sl]))
        out = part if out is None else out + part
    return out
